# Initial kernel scaffold; baseline (speedup 1.0000x reference)
import jax
import jax.numpy as jnp
from jax.experimental import pallas as pl


def kernel(sparse_inputs, tables):
    raise NotImplementedError("write your pallas kernel here")


# speedup vs baseline: 1.1482x; 1.1482x over previous
import math

import jax
import jax.numpy as jnp
from jax import lax
from jax.experimental import pallas as pl
from jax.experimental.pallas import tpu as pltpu
from jax.experimental.pallas import tpu_sc as plsc

F = 26
V = 100000
D = 32
B = 16384
NC, NS, L = 2, 16, 16
NW = NC * NS
N = B * F
RW = N // NW
CH = 128
NCH = RW // CH
PER = (F * CH // math.gcd(F, CH)) // CH
NBUF = 8
GROUPS = NCH // NBUF


def _body(tab, idx, off, out, idx_v, off_v, rows_v, *sems):
    gsem = sems[:NBUF]
    wsem = sems[NBUF:]
    wid = lax.axis_index("s") * NC + lax.axis_index("c")
    base = wid * NCH

    pltpu.sync_copy(idx.at[pl.ds(base, NCH)], idx_v)
    pltpu.sync_copy(off, off_v)

    def adj(j, _):
        jm = lax.rem(j, PER)
        for t in range(CH // L):
            s = pl.ds(t * L, L)
            idx_v[j, s] = idx_v[j, s] + off_v[jm, s]
        return 0

    lax.fori_loop(0, NCH, adj, 0)

    for b in range(NBUF):
        pltpu.async_copy(tab.at[idx_v.at[b]], rows_v.at[b], gsem[b])

    def group(g, _):
        j0 = g * NBUF
        for b in range(NBUF):
            j = j0 + b
            pltpu.make_async_copy(tab.at[idx_v.at[j]], rows_v.at[b], gsem[b]).wait()
            pltpu.async_copy(rows_v.at[b], out.at[pl.ds((base + j) * CH, CH)], wsem[b])
        for b in range(NBUF):
            j = j0 + b
            j2 = j + NBUF
            pltpu.make_async_copy(
                rows_v.at[b], out.at[pl.ds((base + j) * CH, CH)], wsem[b]
            ).wait()

            @pl.when(j2 < NCH)
            def _():
                pltpu.async_copy(tab.at[idx_v.at[j2]], rows_v.at[b], gsem[b])

        return 0

    lax.fori_loop(0, GROUPS, group, 0)


@jax.jit
def kernel(sparse_inputs, tables):
    tab = tables.reshape(F * V, D)
    idx = sparse_inputs.reshape(N // CH, CH)
    off = ((jnp.arange(PER * CH, dtype=jnp.int32) % F) * V).reshape(PER, CH)

    mesh = plsc.VectorSubcoreMesh(
        core_axis_name="c", subcore_axis_name="s", num_cores=NC, num_subcores=NS
    )
    out = pl.kernel(
        _body,
        out_type=jax.ShapeDtypeStruct((N, D), jnp.float32),
        mesh=mesh,
        compiler_params=pltpu.CompilerParams(use_tc_tiling_on_sc=False),
        scratch_types=(
            [
                pltpu.VMEM((NCH, CH), jnp.int32),
                pltpu.VMEM((PER, CH), jnp.int32),
                pltpu.VMEM((NBUF, CH, D), jnp.float32),
            ]
            + [pltpu.SemaphoreType.DMA] * (2 * NBUF)
        ),
    )(tab, idx, off)
    return out.reshape(B, F, D)


# speedup vs baseline: 1.1546x; 1.0056x over previous
import jax
import jax.numpy as jnp
from jax import lax
from jax.experimental import pallas as pl
from jax.experimental.pallas import tpu as pltpu
from jax.experimental.pallas import tpu_sc as plsc

F = 26
V = 100000
D = 32
B = 16384
NC, NS, L = 2, 16, 16
FH = F // NC
BS = B // NS
GCH = 128
NR = FH * BS // GCH
CPF = BS // GCH
NBUF = 8
GRPS = NR // NBUF


def _body(tab, rix, out, idx_v, gbuf, *sems):
    gsem = sems[:NBUF]
    osem = sems[NBUF:]
    c = lax.axis_index("c")
    s = lax.axis_index("s")

    tabf = tab

    pltpu.sync_copy(rix.at[c, pl.ds(s * NR, NR)], idx_v)

    def adj(r, _):
        o = (lax.div(r, CPF) + c * FH) * V
        for t in range(GCH // L):
            sl = pl.ds(t * L, L)
            idx_v[r, sl] = idx_v[r, sl] + o
        return 0

    lax.fori_loop(0, NR, adj, 0)

    def chunk_out(r):
        fl = lax.div(r, CPF)
        b0 = s * BS + lax.rem(r, CPF) * GCH
        return out.at[pl.ds(b0, GCH), c * FH + fl]

    for b in range(NBUF):
        pltpu.async_copy(tabf.at[idx_v.at[b]], gbuf.at[b], gsem[b])

    def group(g, _):
        r0 = g * NBUF
        for b in range(NBUF):
            r = r0 + b
            pltpu.make_async_copy(tabf.at[idx_v.at[r]], gbuf.at[b], gsem[b]).wait()
            pltpu.async_copy(gbuf.at[b], chunk_out(r), osem[b])
        for b in range(NBUF):
            r = r0 + b
            r2 = r + NBUF
            pltpu.make_async_copy(gbuf.at[b], chunk_out(r), osem[b]).wait()

            @pl.when(r2 < NR)
            def _():
                pltpu.async_copy(tabf.at[idx_v.at[r2]], gbuf.at[b], gsem[b])

        return 0

    lax.fori_loop(0, GRPS, group, 0)


@jax.jit
def kernel(sparse_inputs, tables):
    rix = (
        sparse_inputs.reshape(NS, BS, NC, FH)
        .transpose(2, 0, 3, 1)
        .reshape(NC, NS * NR, GCH)
    )

    tab128 = lax.optimization_barrier(tables.reshape(F * V * D // 128, 128))
    tab32 = tab128.reshape(F * V, D)

    mesh = plsc.VectorSubcoreMesh(
        core_axis_name="c", subcore_axis_name="s", num_cores=NC, num_subcores=NS
    )
    out = pl.kernel(
        _body,
        out_type=jax.ShapeDtypeStruct((B, F, D), jnp.float32),
        mesh=mesh,
        compiler_params=pltpu.CompilerParams(use_tc_tiling_on_sc=False),
        scratch_types=(
            [
                pltpu.VMEM((NR, GCH), jnp.int32),
                pltpu.VMEM((NBUF, GCH, D), jnp.float32),
            ]
            + [pltpu.SemaphoreType.DMA] * (2 * NBUF)
        ),
    )(tab32, rix)
    return out


# speedup vs baseline: 3.7746x; 3.2691x over previous
import jax
import jax.numpy as jnp
from jax import lax
from jax.experimental import pallas as pl
from jax.experimental.pallas import tpu as pltpu
from jax.experimental.pallas import tpu_sc as plsc

F = 26
V = 100000
D = 32
B = 16384
NC, NS, L = 2, 16, 16
NW = NC * NS
ROWS = F * D
RPW = ROWS // NW
PC = 2048
NP = B // PC


def _body(tt2, idxt, out2, row_v, ibuf, obuf, rsem, isem, *osem):
    w = lax.axis_index("s") * NC + lax.axis_index("c")

    def do_row(u, _):
        ft = w * RPW + u
        f = lax.div(ft, D)

        pltpu.async_copy(tt2.at[ft], row_v, rsem)
        pltpu.make_async_copy(tt2.at[ft], row_v, rsem).wait()

        def do_pair(p2, _):
            for pb in range(2):
                p = p2 * 2 + pb
                pltpu.async_copy(idxt.at[f, pl.ds(p * PC, PC)], ibuf.at[pb], isem)
                pltpu.make_async_copy(
                    idxt.at[f, pl.ds(p * PC, PC)], ibuf.at[pb], isem
                ).wait()

                @pl.when(jnp.logical_or(p2 >= 1, u > 0))
                def _():
                    pltpu.make_async_copy(
                        obuf.at[pb], out2.at[ft, pl.ds(p * PC, PC)], osem[pb]
                    ).wait()

                def extract(q, _):
                    sl = pl.ds(q * L, L)
                    obuf[pb, sl] = plsc.load_gather(row_v, [ibuf[pb, sl]])
                    return 0

                lax.fori_loop(0, PC // L, extract, 0)
                pltpu.async_copy(
                    obuf.at[pb], out2.at[ft, pl.ds(p * PC, PC)], osem[pb]
                )
            return 0

        lax.fori_loop(0, NP // 2, do_pair, 0)
        return 0

    lax.fori_loop(0, RPW, do_row, 0)

    ftl = w * RPW + RPW - 1
    for pb in range(2):
        pltpu.make_async_copy(
            obuf.at[pb], out2.at[ftl, pl.ds(pb * PC, PC)], osem[pb]
        ).wait()


@jax.jit
def kernel(sparse_inputs, tables):
    tt2 = tables.transpose(0, 2, 1).reshape(ROWS, V)
    idxt = sparse_inputs.T

    mesh = plsc.VectorSubcoreMesh(
        core_axis_name="c", subcore_axis_name="s", num_cores=NC, num_subcores=NS
    )
    out2 = pl.kernel(
        _body,
        out_type=jax.ShapeDtypeStruct((ROWS, B), jnp.float32),
        mesh=mesh,
        compiler_params=pltpu.CompilerParams(
            use_tc_tiling_on_sc=True, needs_layout_passes=False
        ),
        scratch_types=(
            [
                pltpu.VMEM((V,), jnp.float32),
                pltpu.VMEM((2, PC), jnp.int32),
                pltpu.VMEM((2, PC), jnp.float32),
                pltpu.SemaphoreType.DMA,
                pltpu.SemaphoreType.DMA,
            ]
            + [pltpu.SemaphoreType.DMA] * 2
        ),
    )(tt2, idxt)
    return out2.reshape(F, D, B).transpose(2, 0, 1)


# speedup vs baseline: 4.3378x; 1.1492x over previous
import jax
import jax.numpy as jnp
from jax import lax
from jax.experimental import pallas as pl
from jax.experimental.pallas import tpu as pltpu
from jax.experimental.pallas import tpu_sc as plsc

F = 26
V = 100000
D = 32
B = 16384
NC, NS, L = 2, 16, 16
NW = NC * NS
ROWS = F * D
RPW = ROWS // NW
PC = 2048
NP = B // PC


def _body(tt2, idxt, out2, row_v, ibig, obuf, rsem, isem, *osem):
    w = lax.axis_index("s") * NC + lax.axis_index("c")

    def do_row(u, _):
        ft = w * RPW + u
        f = lax.div(ft, D)

        pltpu.async_copy(tt2.at[ft], row_v, rsem)

        @pl.when(jnp.logical_or(u == 0, lax.rem(ft, D) == 0))
        def _():
            pltpu.async_copy(idxt.at[f], ibig, isem)
            pltpu.make_async_copy(idxt.at[f], ibig, isem).wait()

        pltpu.make_async_copy(tt2.at[ft], row_v, rsem).wait()

        def do_pair(p2, _):
            for pb in range(2):
                p = p2 * 2 + pb

                @pl.when(jnp.logical_or(p2 >= 1, u > 0))
                def _():
                    pltpu.make_async_copy(
                        obuf.at[pb], out2.at[ft, pl.ds(p * PC, PC)], osem[pb]
                    ).wait()

                def extract(q, _):
                    for t in range(8):
                        o = q * 8 * L + t * L
                        obuf[pb, pl.ds(o, L)] = plsc.load_gather(
                            row_v, [ibig[pl.ds(p * PC + o, L)]]
                        )
                    return 0

                lax.fori_loop(0, PC // (8 * L), extract, 0)
                pltpu.async_copy(
                    obuf.at[pb], out2.at[ft, pl.ds(p * PC, PC)], osem[pb]
                )
            return 0

        lax.fori_loop(0, NP // 2, do_pair, 0)
        return 0

    lax.fori_loop(0, RPW, do_row, 0)

    ftl = w * RPW + RPW - 1
    for pb in range(2):
        pltpu.make_async_copy(
            obuf.at[pb], out2.at[ftl, pl.ds(pb * PC, PC)], osem[pb]
        ).wait()


@jax.jit
def kernel(sparse_inputs, tables):
    tt2 = tables.transpose(0, 2, 1).reshape(ROWS, V)
    idxt = sparse_inputs.T

    mesh = plsc.VectorSubcoreMesh(
        core_axis_name="c", subcore_axis_name="s", num_cores=NC, num_subcores=NS
    )
    out2 = pl.kernel(
        _body,
        out_type=jax.ShapeDtypeStruct((ROWS, B), jnp.float32),
        mesh=mesh,
        compiler_params=pltpu.CompilerParams(
            use_tc_tiling_on_sc=True, needs_layout_passes=False
        ),
        scratch_types=(
            [
                pltpu.VMEM((V,), jnp.float32),
                pltpu.VMEM((B,), jnp.int32),
                pltpu.VMEM((2, PC), jnp.float32),
                pltpu.SemaphoreType.DMA,
                pltpu.SemaphoreType.DMA,
            ]
            + [pltpu.SemaphoreType.DMA] * 2
        ),
    )(tt2, idxt)
    return out2.reshape(F, D, B).transpose(2, 0, 1)
